# baseline (speedup 1.0000x reference)
import functools

import jax
import jax.numpy as jnp
from jax import lax
from jax.experimental import pallas as pl
from jax.experimental.pallas import tpu as pltpu
from jax.experimental.pallas import tpu_sc as plsc

_VOCAB_TILE = 2048


def _cbow_pool_half_sc(x_half, emb_t):
    Bh, CTX = x_half.shape
    E, V = emb_t.shape
    info = plsc.get_sparse_core_info()
    NC, NS = info.num_cores, info.num_subcores
    NW = NC * NS
    n_tok = (Bh * CTX) // NW
    b_per_w = Bh // NW
    x_t = (x_half.reshape(NW, b_per_w, CTX)
           .transpose(0, 2, 1).reshape(NW, n_tok))

    mesh = plsc.VectorSubcoreMesh(core_axis_name="c", subcore_axis_name="s")

    @functools.partial(
        pl.kernel,
        mesh=mesh,
        compiler_params=pltpu.CompilerParams(use_tc_tiling_on_sc=False),
        out_type=jax.ShapeDtypeStruct((E, Bh), jnp.float32),
        scratch_types=[
            pltpu.VMEM((n_tok,), jnp.int32),
            pltpu.VMEM((E * n_tok,), jnp.float32),
            pltpu.VMEM((E, b_per_w), jnp.float32),
            pltpu.SemaphoreType.DMA,
        ],
    )
    def pool(x_hbm, tab_hbm, out_hbm, xv, rows_v, m_v, sem):
        wid = lax.axis_index("s") * NC + lax.axis_index("c")
        pltpu.sync_copy(x_hbm.at[wid], xv)
        waves = []
        for e in range(E):
            waves.append(pltpu.async_copy(
                tab_hbm.at[e].at[xv],
                rows_v.at[pl.ds(e * n_tok, n_tok)],
                sem,
            ))
            if e >= 3:
                waves[e - 3].wait()
        for c in waves[E - 3:]:
            c.wait()

        scale = jnp.float32(1.0 / CTX)
        n_bg = b_per_w // 16
        for e in range(E):
            for bg in range(n_bg):
                acc = None
                for c in range(CTX):
                    v = rows_v[pl.ds(e * n_tok + c * b_per_w + bg * 16, 16)]
                    acc = v if acc is None else acc + v
                m_v[e, pl.ds(bg * 16, 16)] = acc * scale
        pltpu.sync_copy(m_v, out_hbm.at[:, pl.ds(wid * b_per_w, b_per_w)])

    return pool(x_t, emb_t)


def _project_half_tc(mT, Wt, b2, B, col_blk, prev):
    E, Bh = mT.shape
    V = Wt.shape[1]
    T = _VOCAB_TILE
    n_blk = -(-V // T)

    def body(*refs):
        w_ref, m_ref, b_ref, o_ref = refs[-4:]
        o_ref[...] = lax.dot_general(
            w_ref[...], m_ref[...],
            (((0,), (0,)), ((), ())),
            preferred_element_type=jnp.float32,
        ) + b_ref[...].T

    in_specs = [
        pl.BlockSpec((E, T), lambda i: (0, i)),
        pl.BlockSpec((E, Bh), lambda i: (0, 0)),
        pl.BlockSpec((1, T), lambda i: (0, i)),
    ]
    args = (Wt, mT, b2)
    aliases = {}
    if prev is not None:
        in_specs = [pl.BlockSpec(memory_space=pl.ANY)] + in_specs
        args = (prev,) + args
        aliases = {0: 0}

    return pl.pallas_call(
        body,
        grid=(n_blk,),
        in_specs=in_specs,
        out_specs=pl.BlockSpec((T, Bh), lambda i: (i, col_blk)),
        out_shape=jax.ShapeDtypeStruct((V, B), jnp.float32),
        input_output_aliases=aliases,
    )(*args)


def kernel(x, emb_table, W, b):
    B = x.shape[0]
    V = W.shape[0]
    emb_t = emb_table.T
    Wt = W.T
    b2 = b.reshape(1, V)
    half = B // 2
    mTa = _cbow_pool_half_sc(x[:half], emb_t)
    mTb = _cbow_pool_half_sc(x[half:], emb_t)
    oA = _project_half_tc(mTa, Wt, b2, B, 0, None)
    oB = _project_half_tc(mTb, Wt, b2, B, 1, oA)
    return oB.T


# speedup vs baseline: 1.2660x; 1.2660x over previous
import functools

import jax
import jax.numpy as jnp
from jax import lax
from jax.experimental import pallas as pl
from jax.experimental.pallas import tpu as pltpu
from jax.experimental.pallas import tpu_sc as plsc

_VOCAB_TILE = 2560
_IDX_CHUNK = 640


def _cbow_pool_sc(x, emb_table):
    B, CTX = x.shape
    V, E = emb_table.shape
    info = plsc.get_sparse_core_info()
    NC, NS = info.num_cores, info.num_subcores
    NW = NC * NS
    n_tok = (B * CTX) // NW
    n_ch = n_tok // _IDX_CHUNK
    b_per_w = B // NW
    x_t = x.reshape(NW, b_per_w, CTX).transpose(0, 2, 1).reshape(NW, n_tok)
    emb_t = emb_table.T

    mesh = plsc.VectorSubcoreMesh(core_axis_name="c", subcore_axis_name="s")

    @functools.partial(
        pl.kernel,
        mesh=mesh,
        compiler_params=pltpu.CompilerParams(use_tc_tiling_on_sc=False),
        out_type=jax.ShapeDtypeStruct((E, B), jnp.float32),
        scratch_types=[
            pltpu.VMEM((n_tok,), jnp.int32),
            pltpu.VMEM((E * n_tok,), jnp.float32),
            pltpu.VMEM((E, b_per_w), jnp.float32),
            pltpu.SemaphoreType.DMA,
        ],
    )
    def pool(x_hbm, tab_hbm, out_hbm, xv, rows_v, m_v, sem):
        wid = lax.axis_index("s") * NC + lax.axis_index("c")
        pltpu.sync_copy(x_hbm.at[wid], xv)
        waves = []
        for e in range(E):
            waves.append([
                pltpu.async_copy(
                    tab_hbm.at[e].at[xv.at[pl.ds(g * _IDX_CHUNK, _IDX_CHUNK)]],
                    rows_v.at[pl.ds(e * n_tok + g * _IDX_CHUNK, _IDX_CHUNK)],
                    sem,
                )
                for g in range(n_ch)
            ])
            if e >= 3:
                for c in waves[e - 3]:
                    c.wait()
        for wave in waves[E - 3:]:
            for c in wave:
                c.wait()

        scale = jnp.float32(1.0 / CTX)
        n_bg = b_per_w // 16
        for e in range(E):
            for bg in range(n_bg):
                acc = None
                for c in range(CTX):
                    v = rows_v[pl.ds(e * n_tok + c * b_per_w + bg * 16, 16)]
                    acc = v if acc is None else acc + v
                m_v[e, pl.ds(bg * 16, 16)] = acc * scale
        pltpu.sync_copy(m_v, out_hbm.at[:, pl.ds(wid * b_per_w, b_per_w)])

    return pool(x_t, emb_t)


def _project_tc(mT, W, b):
    E, B = mT.shape
    V = W.shape[0]
    T = _VOCAB_TILE
    n_blk = -(-V // T)
    Wt = W.T
    b2 = b.reshape(1, V)

    def body(w_ref, m_ref, b_ref, o_ref):
        o_ref[...] = lax.dot_general(
            w_ref[...], m_ref[...],
            (((0,), (0,)), ((), ())),
            preferred_element_type=jnp.float32,
        ) + b_ref[...].T

    return pl.pallas_call(
        body,
        grid=(n_blk,),
        in_specs=[
            pl.BlockSpec((E, T), lambda i: (0, i)),
            pl.BlockSpec((E, B), lambda i: (0, 0)),
            pl.BlockSpec((1, T), lambda i: (0, i)),
        ],
        out_specs=pl.BlockSpec((T, B), lambda i: (i, 0)),
        out_shape=jax.ShapeDtypeStruct((V, B), jnp.float32),
    )(Wt, mT, b2)


def kernel(x, emb_table, W, b):
    mT = _cbow_pool_sc(x, emb_table)
    return _project_tc(mT, W, b).T
